# baseline (speedup 1.0000x reference)
import functools

import jax
import jax.numpy as jnp
from jax import lax
from jax.experimental import pallas as pl
from jax.experimental.pallas import tpu as pltpu
from jax.experimental.pallas import tpu_sc as plsc

_TOPK = 2
_EPS = 1e-05
_NC = 2
_NS = 16
_NW = _NC * _NS


def _router_kernel(x_ref, tc_ref, Wt_ref, bt_ref, Wg_ref, pk_ref, cnt_ref, carry_ref):
    t = pl.program_id(0)
    e_cnt = Wg_ref.shape[0]
    tb = x_ref.shape[0]

    @pl.when(t == 0)
    def _():
        carry_ref[...] = jnp.zeros_like(carry_ref)

    x = x_ref[...]
    tc = tc_ref[...]
    xr = x + jnp.dot(tc, Wt_ref[...].T, preferred_element_type=jnp.float32) + bt_ref[...]
    logits = jnp.dot(xr, Wg_ref[...].T, preferred_element_type=jnp.float32)
    p = jax.nn.softmax(logits, axis=-1)
    idx = jax.lax.broadcasted_iota(jnp.int32, p.shape, 1)
    w1 = jnp.max(p, axis=-1, keepdims=True)
    i1 = jnp.argmax(p, axis=-1).reshape(tb, 1)
    mask1 = idx == i1
    p2 = jnp.where(mask1, -jnp.inf, p)
    w2 = jnp.max(p2, axis=-1, keepdims=True)
    i2 = jnp.argmax(p2, axis=-1).reshape(tb, 1)
    mask2 = idx == i2
    denom = w1 + w2 + _EPS

    oh = (mask1 | mask2).astype(jnp.float32)
    r_iota = jax.lax.broadcasted_iota(jnp.int32, (tb, tb), 0)
    c_iota = jax.lax.broadcasted_iota(jnp.int32, (tb, tb), 1)
    ls = (r_iota > c_iota).astype(jnp.float32)
    pre = jnp.dot(ls, oh, preferred_element_type=jnp.float32)
    carry_b = jnp.broadcast_to(carry_ref[...], (tb, e_cnt))
    pre = pre + carry_b
    r1 = jnp.sum(jnp.where(mask1, pre, 0.0), axis=1, keepdims=True)
    r2 = jnp.sum(jnp.where(mask2, pre, 0.0), axis=1, keepdims=True)
    carry_ref[...] += jnp.sum(oh, axis=0, keepdims=True)

    pk_ref[...] = jnp.concatenate(
        [w1 / denom, w2 / denom,
         i1.astype(jnp.float32), i2.astype(jnp.float32),
         r1, r2, jnp.zeros((tb, 2), jnp.float32)], axis=1)
    cnt_ref[...] = carry_ref[...]


def _grouped_kernel(be_ref, xs_ref, Wg_ref, Wu_ref, Wd_ref, o_ref):
    nvalid = be_ref[be_ref.shape[0] - 1]

    @pl.when(pl.program_id(0) < nvalid)
    def _():
        x = xs_ref[...]
        hh = jnp.dot(x, Wg_ref[0].T, preferred_element_type=jnp.float32)
        u = jnp.dot(x, Wu_ref[0].T, preferred_element_type=jnp.float32)
        o_ref[...] = jnp.dot(jax.nn.silu(hh) * u, Wd_ref[0].T,
                             preferred_element_type=jnp.float32)


def _shared_kernel(x_ref, Wg_ref, Wu_ref, Wd_ref, o_ref):
    x = x_ref[...]
    hh = jnp.dot(x, Wg_ref[...].T, preferred_element_type=jnp.float32)
    u = jnp.dot(x, Wu_ref[...].T, preferred_element_type=jnp.float32)
    o_ref[...] = jnp.dot(jax.nn.silu(hh) * u, Wd_ref[...].T,
                         preferred_element_type=jnp.float32)


def _make_dispatch(n, h, cap, nb_pad, g_blk, e_cnt):
    tpw = n // _NW
    ch = min(64, tpw)
    nch = tpw // ch
    lg = g_blk.bit_length() - 1
    mesh = plsc.VectorSubcoreMesh(core_axis_name="c", subcore_axis_name="s")

    @functools.partial(
        pl.kernel,
        out_type=[
            jax.ShapeDtypeStruct((cap, h), jnp.float32),
            jax.ShapeDtypeStruct((n,), jnp.int32),
            jax.ShapeDtypeStruct((n,), jnp.int32),
            jax.ShapeDtypeStruct((nb_pad,), jnp.int32),
        ],
        mesh=mesh,
        scratch_types=[
            pltpu.VMEM((ch, h), jnp.float32),
            pltpu.VMEM((16,), jnp.int32),
            pltpu.VMEM((16,), jnp.int32),
            pltpu.VMEM((16,), jnp.int32),
            pltpu.VMEM((ch,), jnp.int32),
            pltpu.VMEM((ch,), jnp.int32),
            pltpu.VMEM((ch,), jnp.int32),
            pltpu.VMEM((ch,), jnp.int32),
            pltpu.VMEM((nb_pad,), jnp.int32),
        ],
        compiler_params=pltpu.CompilerParams(needs_layout_passes=False),
    )
    def dispatch(x_hbm, e0_hbm, e1_hbm, r0_hbm, r1_hbm, cnt_hbm,
                 xs_hbm, d0_hbm, d1_hbm, be_hbm,
                 xbuf, pd_v, st_v, en_v, d0b, d1b, ev, rv, beb):
        wid = lax.axis_index("s") * _NC + lax.axis_index("c")
        base = wid * tpw

        pltpu.sync_copy(cnt_hbm, pd_v)
        cnt = pd_v[...]
        padded = ((cnt + (g_blk - 1)) >> lg) << lg
        pd_v[...] = padded
        li = lax.iota(jnp.int32, 16)
        ends = jnp.zeros((16,), jnp.int32)
        for e in range(e_cnt):
            eidx = 8 if e == 0 else e
            pe = plsc.load_gather(pd_v, [jnp.full((16,), eidx, jnp.int32)])
            ends = ends + jnp.where(li >= e, pe, 0)
        starts = ends - padded
        st_v[...] = starts
        en_v[...] = ends

        @pl.when(wid == 0)
        def _():
            for grp in range(nb_pad // 16):
                bi = lax.iota(jnp.int32, 16) + grp * 16
                acc = jnp.zeros((16,), jnp.int32)
                for e in range(e_cnt):
                    if e == 0:
                        eb = plsc.load_gather(
                            pd_v, [jnp.full((16,), 8, jnp.int32)]) >> lg
                    else:
                        eb = plsc.load_gather(
                            en_v, [jnp.full((16,), e, jnp.int32)]) >> lg
                    acc += (bi >= eb).astype(jnp.int32)
                beb[pl.ds(grp * 16, 16)] = jnp.minimum(acc, e_cnt - 1)
            pltpu.sync_copy(beb, be_hbm)

        for ci in range(nch):
            cbase = base + ci * ch
            pltpu.sync_copy(x_hbm.at[pl.ds(cbase, ch)], xbuf)
            pltpu.sync_copy(e0_hbm.at[pl.ds(cbase, ch)], ev)
            pltpu.sync_copy(r0_hbm.at[pl.ds(cbase, ch)], rv)
            for i in range(ch // 16):
                sl = pl.ds(i * 16, 16)
                d0b[sl] = plsc.load_gather(st_v, [ev[sl]]) + rv[sl]
            pltpu.sync_copy(d0b, d0_hbm.at[pl.ds(cbase, ch)])
            pltpu.sync_copy(xbuf, xs_hbm.at[d0b])
            pltpu.sync_copy(e1_hbm.at[pl.ds(cbase, ch)], ev)
            pltpu.sync_copy(r1_hbm.at[pl.ds(cbase, ch)], rv)
            for i in range(ch // 16):
                sl = pl.ds(i * 16, 16)
                d1b[sl] = plsc.load_gather(st_v, [ev[sl]]) + rv[sl]
            pltpu.sync_copy(d1b, d1_hbm.at[pl.ds(cbase, ch)])
            pltpu.sync_copy(xbuf, xs_hbm.at[d1b])

    return dispatch


def _make_combine(n, h, cap):
    tpw = n // _NW
    ch = min(32, tpw)
    nch = tpw // ch
    mesh = plsc.VectorSubcoreMesh(core_axis_name="c", subcore_axis_name="s")

    @functools.partial(
        pl.kernel,
        out_type=jax.ShapeDtypeStruct((n, h), jnp.float32),
        mesh=mesh,
        scratch_types=[
            pltpu.VMEM((ch, h), jnp.float32),
            pltpu.VMEM((ch, h), jnp.float32),
            pltpu.VMEM((ch, h), jnp.float32),
            pltpu.VMEM((tpw,), jnp.int32),
            pltpu.VMEM((tpw,), jnp.int32),
            pltpu.VMEM((tpw,), jnp.float32),
            pltpu.VMEM((tpw,), jnp.float32),
            pltpu.SemaphoreType.DMA,
            pltpu.SemaphoreType.DMA,
        ],
        compiler_params=pltpu.CompilerParams(needs_layout_passes=False),
    )
    def combine(ys_hbm, sh_hbm, w0_hbm, w1_hbm, d0_hbm, d1_hbm, out_hbm,
                y0b, y1b, sb, d0a, d1a, w0a, w1a, sem0, sem1):
        wid = lax.axis_index("s") * _NC + lax.axis_index("c")
        base = wid * tpw

        pltpu.sync_copy(d0_hbm.at[pl.ds(base, tpw)], d0a)
        pltpu.sync_copy(d1_hbm.at[pl.ds(base, tpw)], d1a)
        pltpu.sync_copy(w0_hbm.at[pl.ds(base, tpw)], w0a)
        pltpu.sync_copy(w1_hbm.at[pl.ds(base, tpw)], w1a)

        def chunk(ci, carry):
            cps = []
            for i in range(ch // 16):
                isl = pl.ds(ci * ch + i * 16, 16)
                ysl = pl.ds(i * 16, 16)
                cps.append(pltpu.async_copy(
                    ys_hbm.at[d0a[isl]], y0b.at[ysl], sem0))
                cps.append(pltpu.async_copy(
                    ys_hbm.at[d1a[isl]], y1b.at[ysl], sem1))
            pltpu.sync_copy(sh_hbm.at[pl.ds(base + ci * ch, ch)], sb)
            for cp in cps:
                cp.wait()

            def row(r, _):
                ri = jnp.zeros((16,), jnp.int32) + (ci * ch + r)
                wb0 = plsc.load_gather(w0a, [ri])
                wb1 = plsc.load_gather(w1a, [ri])
                for c in range(h // 16):
                    sl = pl.ds(c * 16, 16)
                    y0b[r, sl] = (wb0 * y0b[r, sl]
                                  + wb1 * y1b[r, sl] + sb[r, sl])
                return _

            lax.fori_loop(0, ch, row, 0)
            pltpu.sync_copy(y0b, out_hbm.at[pl.ds(base + ci * ch, ch)])
            return carry

        lax.fori_loop(0, nch, chunk, 0)

    return combine


def kernel(x, temporal_context, Wt, bt, Wg, We_gate, We_up, We_down, Ws_gate, Ws_up, Ws_down):
    b, s, h = x.shape
    n = b * s
    e_cnt, i_dim, _ = We_gate.shape
    x_flat = x.reshape(n, h)
    tc_flat = temporal_context.reshape(n, h)

    t_blk = min(512, n)
    nt = n // t_blk

    packed, counts = pl.pallas_call(
        _router_kernel,
        grid=(nt,),
        in_specs=[
            pl.BlockSpec((t_blk, h), lambda t: (t, 0)),
            pl.BlockSpec((t_blk, h), lambda t: (t, 0)),
            pl.BlockSpec((h, h), lambda t: (0, 0)),
            pl.BlockSpec((1, h), lambda t: (0, 0)),
            pl.BlockSpec((e_cnt, h), lambda t: (0, 0)),
        ],
        out_specs=[
            pl.BlockSpec((t_blk, 8), lambda t: (t, 0)),
            pl.BlockSpec((1, e_cnt), lambda t: (0, 0)),
        ],
        out_shape=[
            jax.ShapeDtypeStruct((n, 8), jnp.float32),
            jax.ShapeDtypeStruct((1, e_cnt), jnp.float32),
        ],
        scratch_shapes=[pltpu.VMEM((1, e_cnt), jnp.float32)],
        compiler_params=pltpu.CompilerParams(
            dimension_semantics=("arbitrary",)),
    )(x_flat, tc_flat, Wt, bt.reshape(1, h), Wg)

    w0 = packed[:, 0]
    w1 = packed[:, 1]
    e0 = packed[:, 2].astype(jnp.int32)
    e1 = packed[:, 3].astype(jnp.int32)
    r0 = packed[:, 4].astype(jnp.int32)
    r1 = packed[:, 5].astype(jnp.int32)
    cnt_i = counts[0].astype(jnp.int32)
    cnt16 = jnp.zeros((16,), jnp.int32).at[:e_cnt].set(cnt_i).at[8].set(cnt_i[0])

    g_blk = 128
    m = n * _TOPK
    cap = m + e_cnt * g_blk
    nb = cap // g_blk
    nb_pad = ((nb + 15) // 16) * 16

    xs, d0, d1, bexp = _make_dispatch(n, h, cap, nb_pad, g_blk, e_cnt)(
        x_flat, e0, e1, r0, r1, cnt16)

    padded_cnt = ((cnt_i + g_blk - 1) // g_blk) * g_blk
    nvalid = (jnp.sum(padded_cnt) // g_blk).astype(jnp.int32)
    be_arr = jnp.concatenate([bexp[:nb], nvalid[None]])

    ys = pl.pallas_call(
        _grouped_kernel,
        grid_spec=pltpu.PrefetchScalarGridSpec(
            num_scalar_prefetch=1,
            grid=(nb,),
            in_specs=[
                pl.BlockSpec((g_blk, h), lambda bb, be: (bb, 0)),
                pl.BlockSpec((1, i_dim, h), lambda bb, be: (be[bb], 0, 0)),
                pl.BlockSpec((1, i_dim, h), lambda bb, be: (be[bb], 0, 0)),
                pl.BlockSpec((1, h, i_dim), lambda bb, be: (be[bb], 0, 0)),
            ],
            out_specs=pl.BlockSpec((g_blk, h), lambda bb, be: (bb, 0)),
        ),
        out_shape=jax.ShapeDtypeStruct((cap, h), jnp.float32),
        compiler_params=pltpu.CompilerParams(
            dimension_semantics=("arbitrary",),
            vmem_limit_bytes=110 * 1024 * 1024,
        ),
    )(be_arr, xs, We_gate, We_up, We_down)

    shared = pl.pallas_call(
        _shared_kernel,
        grid=(nt,),
        in_specs=[
            pl.BlockSpec((t_blk, h), lambda t: (t, 0)),
            pl.BlockSpec((i_dim, h), lambda t: (0, 0)),
            pl.BlockSpec((i_dim, h), lambda t: (0, 0)),
            pl.BlockSpec((h, i_dim), lambda t: (0, 0)),
        ],
        out_specs=pl.BlockSpec((t_blk, h), lambda t: (t, 0)),
        out_shape=jax.ShapeDtypeStruct((n, h), jnp.float32),
        compiler_params=pltpu.CompilerParams(
            dimension_semantics=("arbitrary",),
            vmem_limit_bytes=110 * 1024 * 1024,
        ),
    )(x_flat, Ws_gate, Ws_up, Ws_down)

    out = _make_combine(n, h, cap)(ys, shared, w0, w1, d0, d1)
    return out.reshape(b, s, h)


# speedup vs baseline: 1.5614x; 1.5614x over previous
import functools

import jax
import jax.numpy as jnp
from jax import lax
from jax.experimental import pallas as pl
from jax.experimental.pallas import tpu as pltpu
from jax.experimental.pallas import tpu_sc as plsc

_TOPK = 2
_EPS = 1e-05
_NC = 2
_NS = 16
_NW = _NC * _NS


def _router_kernel(x_ref, tc_ref, Wt_ref, bt_ref, Wg_ref, pk_ref, cnt_ref, carry_ref):
    t = pl.program_id(0)
    e_cnt = Wg_ref.shape[0]
    tb = x_ref.shape[0]

    @pl.when(t == 0)
    def _():
        carry_ref[...] = jnp.zeros_like(carry_ref)

    x = x_ref[...]
    tc = tc_ref[...]
    xr = x + jnp.dot(tc, Wt_ref[...].T, preferred_element_type=jnp.float32) + bt_ref[...]
    logits = jnp.dot(xr, Wg_ref[...].T, preferred_element_type=jnp.float32)
    p = jax.nn.softmax(logits, axis=-1)
    idx = jax.lax.broadcasted_iota(jnp.int32, p.shape, 1)
    w1 = jnp.max(p, axis=-1, keepdims=True)
    i1 = jnp.argmax(p, axis=-1).reshape(tb, 1)
    mask1 = idx == i1
    p2 = jnp.where(mask1, -jnp.inf, p)
    w2 = jnp.max(p2, axis=-1, keepdims=True)
    i2 = jnp.argmax(p2, axis=-1).reshape(tb, 1)
    mask2 = idx == i2
    denom = w1 + w2 + _EPS

    oh = (mask1 | mask2).astype(jnp.float32)
    r_iota = jax.lax.broadcasted_iota(jnp.int32, (tb, tb), 0)
    c_iota = jax.lax.broadcasted_iota(jnp.int32, (tb, tb), 1)
    ls = (r_iota > c_iota).astype(jnp.float32)
    pre = jnp.dot(ls, oh, preferred_element_type=jnp.float32)
    carry_b = jnp.broadcast_to(carry_ref[...], (tb, e_cnt))
    pre = pre + carry_b
    r1 = jnp.sum(jnp.where(mask1, pre, 0.0), axis=1, keepdims=True)
    r2 = jnp.sum(jnp.where(mask2, pre, 0.0), axis=1, keepdims=True)
    carry_ref[...] += jnp.sum(oh, axis=0, keepdims=True)

    pk_ref[...] = jnp.concatenate(
        [w1 / denom, w2 / denom,
         i1.astype(jnp.float32), i2.astype(jnp.float32),
         r1, r2, jnp.zeros((tb, 2), jnp.float32)], axis=1)
    cnt_ref[...] = carry_ref[...]


def _grouped_kernel(be_ref, xs_ref, Wg_ref, Wu_ref, Wd_ref, o_ref):
    nvalid = be_ref[be_ref.shape[0] - 1]

    @pl.when(pl.program_id(0) < nvalid)
    def _():
        x = xs_ref[...]
        hh = jnp.dot(x, Wg_ref[0].T, preferred_element_type=jnp.float32)
        u = jnp.dot(x, Wu_ref[0].T, preferred_element_type=jnp.float32)
        o_ref[...] = jnp.dot(jax.nn.silu(hh) * u, Wd_ref[0].T,
                             preferred_element_type=jnp.float32)


def _shared_kernel(x_ref, Wg_ref, Wu_ref, Wd_ref, o_ref):
    x = x_ref[...]
    hh = jnp.dot(x, Wg_ref[...].T, preferred_element_type=jnp.float32)
    u = jnp.dot(x, Wu_ref[...].T, preferred_element_type=jnp.float32)
    o_ref[...] = jnp.dot(jax.nn.silu(hh) * u, Wd_ref[...].T,
                         preferred_element_type=jnp.float32)


def _make_dispatch(n, h, cap, nb_pad, g_blk, e_cnt):
    tpw = n // _NW
    ch = min(64, tpw)
    nch = tpw // ch
    lg = g_blk.bit_length() - 1
    mesh = plsc.VectorSubcoreMesh(core_axis_name="c", subcore_axis_name="s")

    @functools.partial(
        pl.kernel,
        out_type=[
            jax.ShapeDtypeStruct((cap, h), jnp.float32),
            jax.ShapeDtypeStruct((n,), jnp.int32),
            jax.ShapeDtypeStruct((n,), jnp.int32),
            jax.ShapeDtypeStruct((nb_pad,), jnp.int32),
        ],
        mesh=mesh,
        scratch_types=[
            pltpu.VMEM((ch, h), jnp.float32),
            pltpu.VMEM((16,), jnp.int32),
            pltpu.VMEM((16,), jnp.int32),
            pltpu.VMEM((16,), jnp.int32),
            pltpu.VMEM((ch,), jnp.int32),
            pltpu.VMEM((ch,), jnp.int32),
            pltpu.VMEM((ch,), jnp.int32),
            pltpu.VMEM((ch,), jnp.int32),
            pltpu.VMEM((nb_pad,), jnp.int32),
        ],
        compiler_params=pltpu.CompilerParams(needs_layout_passes=False),
    )
    def dispatch(x_hbm, e0_hbm, e1_hbm, r0_hbm, r1_hbm, cnt_hbm,
                 xs_hbm, d0_hbm, d1_hbm, be_hbm,
                 xbuf, pd_v, st_v, en_v, d0b, d1b, ev, rv, beb):
        wid = lax.axis_index("s") * _NC + lax.axis_index("c")
        base = wid * tpw

        pltpu.sync_copy(cnt_hbm, pd_v)
        cnt = pd_v[...]
        padded = ((cnt + (g_blk - 1)) >> lg) << lg
        pd_v[...] = padded
        li = lax.iota(jnp.int32, 16)
        ends = jnp.zeros((16,), jnp.int32)
        for e in range(e_cnt):
            eidx = 8 if e == 0 else e
            pe = plsc.load_gather(pd_v, [jnp.full((16,), eidx, jnp.int32)])
            ends = ends + jnp.where(li >= e, pe, 0)
        starts = ends - padded
        st_v[...] = starts
        en_v[...] = ends

        @pl.when(wid == 0)
        def _():
            for grp in range(nb_pad // 16):
                bi = lax.iota(jnp.int32, 16) + grp * 16
                acc = jnp.zeros((16,), jnp.int32)
                for e in range(e_cnt):
                    if e == 0:
                        eb = plsc.load_gather(
                            pd_v, [jnp.full((16,), 8, jnp.int32)]) >> lg
                    else:
                        eb = plsc.load_gather(
                            en_v, [jnp.full((16,), e, jnp.int32)]) >> lg
                    acc += (bi >= eb).astype(jnp.int32)
                beb[pl.ds(grp * 16, 16)] = jnp.minimum(acc, e_cnt - 1)
            pltpu.sync_copy(beb, be_hbm)

        for ci in range(nch):
            cbase = base + ci * ch
            pltpu.sync_copy(x_hbm.at[pl.ds(cbase, ch)], xbuf)
            pltpu.sync_copy(e0_hbm.at[pl.ds(cbase, ch)], ev)
            pltpu.sync_copy(r0_hbm.at[pl.ds(cbase, ch)], rv)
            for i in range(ch // 16):
                sl = pl.ds(i * 16, 16)
                d0b[sl] = plsc.load_gather(st_v, [ev[sl]]) + rv[sl]
            pltpu.sync_copy(d0b, d0_hbm.at[pl.ds(cbase, ch)])
            pltpu.sync_copy(xbuf, xs_hbm.at[d0b])
            pltpu.sync_copy(e1_hbm.at[pl.ds(cbase, ch)], ev)
            pltpu.sync_copy(r1_hbm.at[pl.ds(cbase, ch)], rv)
            for i in range(ch // 16):
                sl = pl.ds(i * 16, 16)
                d1b[sl] = plsc.load_gather(st_v, [ev[sl]]) + rv[sl]
            pltpu.sync_copy(d1b, d1_hbm.at[pl.ds(cbase, ch)])
            pltpu.sync_copy(xbuf, xs_hbm.at[d1b])

    return dispatch


def _make_combine(n, h, cap):
    tpw = n // _NW
    ch = min(32, tpw)
    nch = tpw // ch
    mesh = plsc.VectorSubcoreMesh(core_axis_name="c", subcore_axis_name="s")

    @functools.partial(
        pl.kernel,
        out_type=jax.ShapeDtypeStruct((n, h), jnp.float32),
        mesh=mesh,
        scratch_types=[
            pltpu.VMEM((ch, h), jnp.float32),
            pltpu.VMEM((ch, h), jnp.float32),
            pltpu.VMEM((ch, h), jnp.float32),
            pltpu.VMEM((tpw,), jnp.int32),
            pltpu.VMEM((tpw,), jnp.int32),
            pltpu.VMEM((tpw,), jnp.float32),
            pltpu.VMEM((tpw,), jnp.float32),
            pltpu.SemaphoreType.DMA,
            pltpu.SemaphoreType.DMA,
        ],
        compiler_params=pltpu.CompilerParams(needs_layout_passes=False),
    )
    def combine(ys_hbm, sh_hbm, w0_hbm, w1_hbm, d0_hbm, d1_hbm, out_hbm,
                y0b, y1b, sb, d0a, d1a, w0a, w1a, sem0, sem1):
        wid = lax.axis_index("s") * _NC + lax.axis_index("c")
        base = wid * tpw

        pltpu.sync_copy(d0_hbm.at[pl.ds(base, tpw)], d0a)
        pltpu.sync_copy(d1_hbm.at[pl.ds(base, tpw)], d1a)
        pltpu.sync_copy(w0_hbm.at[pl.ds(base, tpw)], w0a)
        pltpu.sync_copy(w1_hbm.at[pl.ds(base, tpw)], w1a)

        def chunk(ci, carry):
            cps = []
            for i in range(ch // 16):
                isl = pl.ds(ci * ch + i * 16, 16)
                ysl = pl.ds(i * 16, 16)
                cps.append(pltpu.async_copy(
                    ys_hbm.at[d0a[isl]], y0b.at[ysl], sem0))
                cps.append(pltpu.async_copy(
                    ys_hbm.at[d1a[isl]], y1b.at[ysl], sem1))
            pltpu.sync_copy(sh_hbm.at[pl.ds(base + ci * ch, ch)], sb)
            for cp in cps:
                cp.wait()

            def row(r, _):
                ri = jnp.zeros((16,), jnp.int32) + (ci * ch + r)
                wb0 = plsc.load_gather(w0a, [ri])
                wb1 = plsc.load_gather(w1a, [ri])
                for c in range(h // 16):
                    sl = pl.ds(c * 16, 16)
                    y0b[r, sl] = (wb0 * y0b[r, sl]
                                  + wb1 * y1b[r, sl] + sb[r, sl])
                return _

            lax.fori_loop(0, ch, row, 0)
            pltpu.sync_copy(y0b, out_hbm.at[pl.ds(base + ci * ch, ch)])
            return carry

        lax.fori_loop(0, nch, chunk, 0)

    return combine


def kernel(x, temporal_context, Wt, bt, Wg, We_gate, We_up, We_down, Ws_gate, Ws_up, Ws_down):
    b, s, h = x.shape
    n = b * s
    e_cnt, i_dim, _ = We_gate.shape
    x_flat = x.reshape(n, h)
    tc_flat = temporal_context.reshape(n, h)

    t_blk = min(512, n)
    nt = n // t_blk

    packed, counts = pl.pallas_call(
        _router_kernel,
        grid=(nt,),
        in_specs=[
            pl.BlockSpec((t_blk, h), lambda t: (t, 0)),
            pl.BlockSpec((t_blk, h), lambda t: (t, 0)),
            pl.BlockSpec((h, h), lambda t: (0, 0)),
            pl.BlockSpec((1, h), lambda t: (0, 0)),
            pl.BlockSpec((e_cnt, h), lambda t: (0, 0)),
        ],
        out_specs=[
            pl.BlockSpec((t_blk, 8), lambda t: (t, 0)),
            pl.BlockSpec((1, e_cnt), lambda t: (0, 0)),
        ],
        out_shape=[
            jax.ShapeDtypeStruct((n, 8), jnp.float32),
            jax.ShapeDtypeStruct((1, e_cnt), jnp.float32),
        ],
        scratch_shapes=[pltpu.VMEM((1, e_cnt), jnp.float32)],
        compiler_params=pltpu.CompilerParams(
            dimension_semantics=("arbitrary",)),
    )(x_flat, tc_flat, Wt, bt.reshape(1, h), Wg)

    w0 = packed[:, 0]
    w1 = packed[:, 1]
    e0 = packed[:, 2].astype(jnp.int32)
    e1 = packed[:, 3].astype(jnp.int32)
    r0 = packed[:, 4].astype(jnp.int32)
    r1 = packed[:, 5].astype(jnp.int32)
    cnt_i = counts[0].astype(jnp.int32)
    cnt16 = jnp.zeros((16,), jnp.int32).at[:e_cnt].set(cnt_i).at[8].set(cnt_i[0])

    g_blk = 512
    m = n * _TOPK
    cap = m + e_cnt * g_blk
    nb = cap // g_blk
    nb_pad = ((nb + 15) // 16) * 16

    xs, d0, d1, bexp = _make_dispatch(n, h, cap, nb_pad, g_blk, e_cnt)(
        x_flat, e0, e1, r0, r1, cnt16)

    padded_cnt = ((cnt_i + g_blk - 1) // g_blk) * g_blk
    nvalid = (jnp.sum(padded_cnt) // g_blk).astype(jnp.int32)
    be_arr = jnp.concatenate([bexp[:nb], nvalid[None]])

    ys = pl.pallas_call(
        _grouped_kernel,
        grid_spec=pltpu.PrefetchScalarGridSpec(
            num_scalar_prefetch=1,
            grid=(nb,),
            in_specs=[
                pl.BlockSpec((g_blk, h), lambda bb, be: (bb, 0)),
                pl.BlockSpec((1, i_dim, h), lambda bb, be: (be[bb], 0, 0)),
                pl.BlockSpec((1, i_dim, h), lambda bb, be: (be[bb], 0, 0)),
                pl.BlockSpec((1, h, i_dim), lambda bb, be: (be[bb], 0, 0)),
            ],
            out_specs=pl.BlockSpec((g_blk, h), lambda bb, be: (bb, 0)),
        ),
        out_shape=jax.ShapeDtypeStruct((cap, h), jnp.float32),
        compiler_params=pltpu.CompilerParams(
            dimension_semantics=("arbitrary",),
            vmem_limit_bytes=110 * 1024 * 1024,
        ),
    )(be_arr, xs, We_gate, We_up, We_down)

    shared = pl.pallas_call(
        _shared_kernel,
        grid=(nt,),
        in_specs=[
            pl.BlockSpec((t_blk, h), lambda t: (t, 0)),
            pl.BlockSpec((i_dim, h), lambda t: (0, 0)),
            pl.BlockSpec((i_dim, h), lambda t: (0, 0)),
            pl.BlockSpec((h, i_dim), lambda t: (0, 0)),
        ],
        out_specs=pl.BlockSpec((t_blk, h), lambda t: (t, 0)),
        out_shape=jax.ShapeDtypeStruct((n, h), jnp.float32),
        compiler_params=pltpu.CompilerParams(
            dimension_semantics=("arbitrary",),
            vmem_limit_bytes=110 * 1024 * 1024,
        ),
    )(x_flat, Ws_gate, Ws_up, Ws_down)

    out = _make_combine(n, h, cap)(ys, shared, w0, w1, d0, d1)
    return out.reshape(b, s, h)
